# initial kernel scaffold (unmeasured)
import jax
import jax.numpy as jnp
from jax import lax
from jax.experimental import pallas as pl
from jax.experimental.pallas import tpu as pltpu

N_DEV = 32
B_LOC = 2
HQ_LOC = 4
DH = 64
SQ = 128
SKV = 128
DM = 512
HD_LOC = HQ_LOC * DH
BLK = 64


def kernel(x, Wq, K_ext, V_ext, Wo):
    my = lax.axis_index("i")

    kb = lax.dynamic_slice_in_dim(K_ext, my * B_LOC, B_LOC, axis=0)
    vb = lax.dynamic_slice_in_dim(V_ext, my * B_LOC, B_LOC, axis=0)
    k_t = jnp.transpose(kb.astype(jnp.bfloat16), (0, 2, 3, 1))
    v_t = jnp.transpose(vb.astype(jnp.bfloat16), (0, 2, 1, 3))
    x_flat = x.reshape(B_LOC * SQ, DM).astype(jnp.bfloat16)
    c0 = jnp.concatenate(
        [Wq.astype(jnp.bfloat16), Wo.astype(jnp.bfloat16).T], axis=1
    )

    def body(x_ref, c0_ref, k_ref, v_ref, out_ref,
             comm_ref, send_sems, recv_sems, acc_ref):
        my_pos = lax.axis_index("i")
        left = lax.rem(my_pos - 1 + N_DEV, N_DEV)
        right = lax.rem(my_pos + 1, N_DEV)

        barrier_sem = pltpu.get_barrier_semaphore()
        for nbr in (left, right):
            pl.semaphore_signal(
                barrier_sem, inc=1,
                device_id=(nbr,), device_id_type=pl.DeviceIdType.MESH,
            )
        pl.semaphore_wait(barrier_sem, 2)

        qb = lax.broadcasted_iota(jnp.int32, (SQ, SKV), 0) // BLK
        kbi = lax.broadcasted_iota(jnp.int32, (SQ, SKV), 1) // BLK
        mask = (kbi <= qb)[None, None, :, :]

        def contribution(chunk, origin):
            wq = chunk[:, :HD_LOC]
            wot = chunk[:, HD_LOC:]
            q = jnp.dot(x_ref[...], wq, preferred_element_type=jnp.float32)
            q = (q * 0.125).astype(jnp.bfloat16)
            q = q.reshape(B_LOC, SQ, HQ_LOC, DH).transpose(0, 2, 1, 3)
            h0 = origin * HQ_LOC
            k = k_ref[:, pl.ds(h0, HQ_LOC), :, :]
            v = v_ref[:, pl.ds(h0, HQ_LOC), :, :]
            s = jnp.einsum("bhid,bhdj->bhij", q, k,
                           preferred_element_type=jnp.float32)
            s = jnp.where(mask, s, -1e9)
            m = jnp.max(s, axis=-1, keepdims=True)
            w = jnp.exp(s - m)
            w = w / jnp.sum(w, axis=-1, keepdims=True)
            ctx = jnp.einsum("bhij,bhjd->bhid", w.astype(jnp.bfloat16), v,
                             preferred_element_type=jnp.float32)
            ctx = ctx.astype(jnp.bfloat16).transpose(0, 2, 1, 3)
            ctx = ctx.reshape(B_LOC * SQ, HD_LOC)
            return lax.dot_general(
                ctx, wot, (((1,), (1,)), ((), ())),
                preferred_element_type=jnp.float32,
            )

        acc_ref[...] = contribution(c0_ref[...], my_pos)
        comm_ref[0] = c0_ref[...]

        def hop(h, _):
            s_slot = lax.rem(h, 2)
            r_slot = lax.rem(h + 1, 2)
            rdma = pltpu.make_async_remote_copy(
                src_ref=comm_ref.at[s_slot],
                dst_ref=comm_ref.at[r_slot],
                send_sem=send_sems.at[s_slot],
                recv_sem=recv_sems.at[r_slot],
                device_id=(right,),
                device_id_type=pl.DeviceIdType.MESH,
            )
            rdma.start()
            rdma.wait()
            origin = lax.rem(my_pos - h - 1 + 2 * N_DEV, N_DEV)
            acc_ref[...] += contribution(comm_ref[r_slot], origin)
            return 0

        lax.fori_loop(0, N_DEV - 1, hop, 0)

        out_ref[...] = acc_ref[...].reshape(B_LOC, SQ, DM)

    return pl.pallas_call(
        body,
        out_shape=jax.ShapeDtypeStruct((B_LOC, SQ, DM), jnp.float32),
        in_specs=[
            pl.BlockSpec(memory_space=pltpu.VMEM),
            pl.BlockSpec(memory_space=pltpu.VMEM),
            pl.BlockSpec(memory_space=pltpu.VMEM),
            pl.BlockSpec(memory_space=pltpu.VMEM),
        ],
        out_specs=pl.BlockSpec(memory_space=pltpu.VMEM),
        scratch_shapes=[
            pltpu.VMEM((2, DM, 2 * HD_LOC), jnp.bfloat16),
            pltpu.SemaphoreType.DMA((2,)),
            pltpu.SemaphoreType.DMA((2,)),
            pltpu.VMEM((B_LOC * SQ, DM), jnp.float32),
        ],
        compiler_params=pltpu.CompilerParams(collective_id=0),
    )(x_flat, c0, k_t, v_t)


# baseline (device time: 303353 ns/iter reference)
import jax
import jax.numpy as jnp
from jax import lax
from jax.experimental import pallas as pl
from jax.experimental.pallas import tpu as pltpu

N_DEV = 32
B_LOC = 2
HQ_LOC = 4
DH = 64
SQ = 128
SKV = 128
DM = 512
HD_LOC = HQ_LOC * DH
BLK = 64


def kernel(x, Wq, K_ext, V_ext, Wo):
    my = lax.axis_index("i")

    kb = lax.dynamic_slice_in_dim(K_ext, my * B_LOC, B_LOC, axis=0)
    vb = lax.dynamic_slice_in_dim(V_ext, my * B_LOC, B_LOC, axis=0)
    k_t = jnp.transpose(kb.astype(jnp.bfloat16), (0, 2, 3, 1))
    v_t = jnp.transpose(vb.astype(jnp.bfloat16), (0, 2, 1, 3))
    x_flat = x.reshape(B_LOC * SQ, DM).astype(jnp.bfloat16)
    c0 = jnp.concatenate(
        [Wq.astype(jnp.bfloat16), Wo.astype(jnp.bfloat16).T], axis=1
    )

    def body(x_ref, c0_ref, k_ref, v_ref, out_ref,
             comm_ref, send_sems, recv_sems, acc_ref):
        my_pos = lax.axis_index("i")
        left = lax.rem(my_pos - 1 + N_DEV, N_DEV)
        right = lax.rem(my_pos + 1, N_DEV)

        barrier_sem = pltpu.get_barrier_semaphore()
        for nbr in (left, right):
            pl.semaphore_signal(
                barrier_sem, inc=1,
                device_id=(nbr,), device_id_type=pl.DeviceIdType.MESH,
            )
        pl.semaphore_wait(barrier_sem, 2)

        qb = lax.broadcasted_iota(jnp.int32, (SQ, SKV), 0) // BLK
        kbi = lax.broadcasted_iota(jnp.int32, (SQ, SKV), 1) // BLK
        mask = (kbi <= qb)[None, :, :]

        def contribution(chunk, origin):
            wq = chunk[:, :HD_LOC]
            wot = chunk[:, HD_LOC:]
            q = jnp.dot(x_ref[...], wq, preferred_element_type=jnp.float32)
            q = (q * 0.125).astype(jnp.bfloat16)
            q = q.reshape(B_LOC, SQ, HQ_LOC, DH).transpose(0, 2, 1, 3)
            q = q.reshape(B_LOC * HQ_LOC, SQ, DH)
            h0 = origin * HQ_LOC
            k = k_ref[:, pl.ds(h0, HQ_LOC), :, :].reshape(
                B_LOC * HQ_LOC, DH, SKV)
            v = v_ref[:, pl.ds(h0, HQ_LOC), :, :].reshape(
                B_LOC * HQ_LOC, SKV, DH)
            s = jnp.einsum("gid,gdj->gij", q, k,
                           preferred_element_type=jnp.float32)
            s = jnp.where(mask, s, -1e9)
            m = jnp.max(s, axis=-1, keepdims=True)
            w = jnp.exp(s - m)
            w = w / jnp.sum(w, axis=-1, keepdims=True)
            ctx = jnp.einsum("gij,gjd->gid", w.astype(jnp.bfloat16), v,
                             preferred_element_type=jnp.float32)
            ctx = ctx.astype(jnp.bfloat16).reshape(B_LOC, HQ_LOC, SQ, DH)
            ctx = ctx.transpose(0, 2, 1, 3).reshape(B_LOC * SQ, HD_LOC)
            return lax.dot_general(
                ctx, wot, (((1,), (1,)), ((), ())),
                preferred_element_type=jnp.float32,
            )

        acc_ref[...] = contribution(c0_ref[...], my_pos)
        comm_ref[0] = c0_ref[...]

        def hop(h, _):
            s_slot = lax.rem(h, 2)
            r_slot = lax.rem(h + 1, 2)
            rdma = pltpu.make_async_remote_copy(
                src_ref=comm_ref.at[s_slot],
                dst_ref=comm_ref.at[r_slot],
                send_sem=send_sems.at[s_slot],
                recv_sem=recv_sems.at[r_slot],
                device_id=(right,),
                device_id_type=pl.DeviceIdType.MESH,
            )
            rdma.start()
            rdma.wait()
            origin = lax.rem(my_pos - h - 1 + 2 * N_DEV, N_DEV)
            acc_ref[...] += contribution(comm_ref[r_slot], origin)
            return 0

        lax.fori_loop(0, N_DEV - 1, hop, 0)

        out_ref[...] = acc_ref[...].reshape(B_LOC, SQ, DM)

    return pl.pallas_call(
        body,
        out_shape=jax.ShapeDtypeStruct((B_LOC, SQ, DM), jnp.float32),
        in_specs=[
            pl.BlockSpec(memory_space=pltpu.VMEM),
            pl.BlockSpec(memory_space=pltpu.VMEM),
            pl.BlockSpec(memory_space=pltpu.VMEM),
            pl.BlockSpec(memory_space=pltpu.VMEM),
        ],
        out_specs=pl.BlockSpec(memory_space=pltpu.VMEM),
        scratch_shapes=[
            pltpu.VMEM((2, DM, 2 * HD_LOC), jnp.bfloat16),
            pltpu.SemaphoreType.DMA((2,)),
            pltpu.SemaphoreType.DMA((2,)),
            pltpu.VMEM((B_LOC * SQ, DM), jnp.float32),
        ],
        compiler_params=pltpu.CompilerParams(collective_id=0),
    )(x_flat, c0, k_t, v_t)


# device time: 223768 ns/iter; 1.3557x vs baseline; 1.3557x over previous
import jax
import jax.numpy as jnp
from jax import lax
from jax.experimental import pallas as pl
from jax.experimental.pallas import tpu as pltpu

N_DEV = 32
N_CW = 16
N_CCW = 15
B_LOC = 2
HQ_LOC = 4
DH = 64
SQ = 128
SKV = 128
DM = 512
HD_LOC = HQ_LOC * DH
BLK = 64


def kernel(x, Wq, K_ext, V_ext, Wo):
    my = lax.axis_index("i")

    kb = lax.dynamic_slice_in_dim(K_ext, my * B_LOC, B_LOC, axis=0)
    vb = lax.dynamic_slice_in_dim(V_ext, my * B_LOC, B_LOC, axis=0)
    k_t = jnp.transpose(kb.astype(jnp.bfloat16), (0, 2, 3, 1))
    v_t = jnp.transpose(vb.astype(jnp.bfloat16), (0, 2, 1, 3))
    x_flat = x.reshape(B_LOC * SQ, DM).astype(jnp.bfloat16)
    c0 = jnp.concatenate(
        [Wq.astype(jnp.bfloat16), Wo.astype(jnp.bfloat16).T], axis=1
    )

    def body(x_ref, c0_ref, k_ref, v_ref, out_ref,
             cw_buf, ccw_buf, cw_send, cw_recv, ccw_send, ccw_recv, acc_ref):
        my_pos = lax.axis_index("i")
        left = lax.rem(my_pos - 1 + N_DEV, N_DEV)
        right = lax.rem(my_pos + 1, N_DEV)

        barrier_sem = pltpu.get_barrier_semaphore()
        for nbr in (left, right):
            pl.semaphore_signal(
                barrier_sem, inc=1,
                device_id=(nbr,), device_id_type=pl.DeviceIdType.MESH,
            )
        pl.semaphore_wait(barrier_sem, 2)

        qb = lax.broadcasted_iota(jnp.int32, (SQ, SKV), 0) // BLK
        kbi = lax.broadcasted_iota(jnp.int32, (SQ, SKV), 1) // BLK
        mask = (kbi <= qb)[None, :, :]

        def contribution(chunk, origin):
            wq = chunk[:, :HD_LOC]
            wot = chunk[:, HD_LOC:]
            q = jnp.dot(x_ref[...], wq, preferred_element_type=jnp.float32)
            q = (q * 0.125).astype(jnp.bfloat16)
            q = q.reshape(B_LOC, SQ, HQ_LOC, DH).transpose(0, 2, 1, 3)
            q = q.reshape(B_LOC * HQ_LOC, SQ, DH)
            h0 = origin * HQ_LOC
            k = k_ref[:, pl.ds(h0, HQ_LOC), :, :].reshape(
                B_LOC * HQ_LOC, DH, SKV)
            v = v_ref[:, pl.ds(h0, HQ_LOC), :, :].reshape(
                B_LOC * HQ_LOC, SKV, DH)
            s = jnp.einsum("gid,gdj->gij", q, k,
                           preferred_element_type=jnp.float32)
            s = jnp.where(mask, s, -1e9)
            m = jnp.max(s, axis=-1, keepdims=True)
            w = jnp.exp(s - m)
            w = w / jnp.sum(w, axis=-1, keepdims=True)
            ctx = jnp.einsum("gij,gjd->gid", w.astype(jnp.bfloat16), v,
                             preferred_element_type=jnp.float32)
            ctx = ctx.astype(jnp.bfloat16).reshape(B_LOC, HQ_LOC, SQ, DH)
            ctx = ctx.transpose(0, 2, 1, 3).reshape(B_LOC * SQ, HD_LOC)
            return lax.dot_general(
                ctx, wot, (((1,), (1,)), ((), ())),
                preferred_element_type=jnp.float32,
            )

        def fwd(src, dst_buf, slot, send_sems, recv_sems, tgt):
            return pltpu.make_async_remote_copy(
                src_ref=src,
                dst_ref=dst_buf.at[slot],
                send_sem=send_sems.at[slot],
                recv_sem=recv_sems.at[slot],
                device_id=(tgt,),
                device_id_type=pl.DeviceIdType.MESH,
            )

        fwd(c0_ref, cw_buf, 0, cw_send, cw_recv, right).start()
        fwd(c0_ref, ccw_buf, 0, ccw_send, ccw_recv, left).start()
        acc_ref[...] = contribution(c0_ref[...], my_pos)

        def step(h, _):
            fwd(c0_ref, cw_buf, h, cw_send, cw_recv, right).wait_recv()

            @pl.when(h < N_CW - 1)
            def _():
                fwd(cw_buf.at[h], cw_buf, h + 1, cw_send, cw_recv,
                    right).start()

            fwd(c0_ref, ccw_buf, h, ccw_send, ccw_recv, left).wait_recv()

            @pl.when(h < N_CCW - 1)
            def _():
                fwd(ccw_buf.at[h], ccw_buf, h + 1, ccw_send, ccw_recv,
                    left).start()

            cw_origin = lax.rem(my_pos - 1 - h + 2 * N_DEV, N_DEV)
            ccw_origin = lax.rem(my_pos + 1 + h, N_DEV)
            acc_ref[...] += contribution(cw_buf[h], cw_origin)
            acc_ref[...] += contribution(ccw_buf[h], ccw_origin)

            fwd(cw_buf.at[h], cw_buf, h, cw_send, cw_recv, right).wait_send()
            fwd(ccw_buf.at[h], ccw_buf, h, ccw_send, ccw_recv,
                left).wait_send()
            return 0

        lax.fori_loop(0, N_CCW, step, 0)

        fwd(c0_ref, cw_buf, N_CW - 1, cw_send, cw_recv, right).wait_recv()
        fwd(cw_buf.at[N_CW - 1], cw_buf, N_CW - 1, cw_send, cw_recv,
            right).wait_send()
        far = lax.rem(my_pos + N_CW, N_DEV)
        acc_ref[...] += contribution(cw_buf[N_CW - 1], far)

        out_ref[...] = acc_ref[...].reshape(B_LOC, SQ, DM)

    return pl.pallas_call(
        body,
        out_shape=jax.ShapeDtypeStruct((B_LOC, SQ, DM), jnp.float32),
        in_specs=[
            pl.BlockSpec(memory_space=pltpu.VMEM),
            pl.BlockSpec(memory_space=pltpu.VMEM),
            pl.BlockSpec(memory_space=pltpu.VMEM),
            pl.BlockSpec(memory_space=pltpu.VMEM),
        ],
        out_specs=pl.BlockSpec(memory_space=pltpu.VMEM),
        scratch_shapes=[
            pltpu.VMEM((N_CW, DM, 2 * HD_LOC), jnp.bfloat16),
            pltpu.VMEM((N_CCW, DM, 2 * HD_LOC), jnp.bfloat16),
            pltpu.SemaphoreType.DMA((N_CW,)),
            pltpu.SemaphoreType.DMA((N_CW,)),
            pltpu.SemaphoreType.DMA((N_CCW,)),
            pltpu.SemaphoreType.DMA((N_CCW,)),
            pltpu.VMEM((B_LOC * SQ, DM), jnp.float32),
        ],
        compiler_params=pltpu.CompilerParams(collective_id=0),
    )(x_flat, c0, k_t, v_t)


# device time: 215534 ns/iter; 1.4074x vs baseline; 1.0382x over previous
import jax
import jax.numpy as jnp
from jax import lax
from jax.experimental import pallas as pl
from jax.experimental.pallas import tpu as pltpu

N_DEV = 32
N_CW = 16
N_CCW = 15
B_LOC = 2
HQ_LOC = 4
DH = 64
SQ = 128
SKV = 128
DM = 512
HD_LOC = HQ_LOC * DH
BLK = 64


def kernel(x, Wq, K_ext, V_ext, Wo):
    my = lax.axis_index("i")

    kb = lax.dynamic_slice_in_dim(K_ext, my * B_LOC, B_LOC, axis=0)
    vb = lax.dynamic_slice_in_dim(V_ext, my * B_LOC, B_LOC, axis=0)
    k_t = jnp.transpose(kb.astype(jnp.bfloat16), (0, 2, 3, 1))
    v_t = jnp.transpose(vb.astype(jnp.bfloat16), (0, 2, 1, 3))
    x_flat = x.reshape(B_LOC * SQ, DM).astype(jnp.bfloat16)
    c0 = jnp.concatenate(
        [Wq.astype(jnp.bfloat16), Wo.astype(jnp.bfloat16).T], axis=1
    )

    def body(x_ref, c0_ref, k_ref, v_ref, out_ref,
             cw_buf, ccw_buf, cw_send, cw_recv, ccw_send, ccw_recv, acc_ref):
        my_pos = lax.axis_index("i")
        left = lax.rem(my_pos - 1 + N_DEV, N_DEV)
        right = lax.rem(my_pos + 1, N_DEV)

        barrier_sem = pltpu.get_barrier_semaphore()
        for nbr in (left, right):
            pl.semaphore_signal(
                barrier_sem, inc=1,
                device_id=(nbr,), device_id_type=pl.DeviceIdType.MESH,
            )
        pl.semaphore_wait(barrier_sem, 2)

        def contribution(chunk, origin):
            wq = chunk[:, :HD_LOC]
            wot = chunk[:, HD_LOC:]
            q = jnp.dot(x_ref[...], wq, preferred_element_type=jnp.float32)
            q = (q * 0.125).astype(jnp.bfloat16)
            q = q.reshape(B_LOC, SQ, HQ_LOC, DH).transpose(0, 2, 1, 3)
            q = q.reshape(B_LOC * HQ_LOC, SQ, DH)
            h0 = origin * HQ_LOC
            k = k_ref[:, pl.ds(h0, HQ_LOC), :, :].reshape(
                B_LOC * HQ_LOC, DH, SKV)
            v = v_ref[:, pl.ds(h0, HQ_LOC), :, :].reshape(
                B_LOC * HQ_LOC, SKV, DH)
            q0 = q[:, :BLK, :]
            q1 = q[:, BLK:, :]
            k0 = k[:, :, :BLK]
            v0 = v[:, :BLK, :]
            e0 = jnp.exp(jnp.einsum("gid,gdj->gij", q0, k0,
                                    preferred_element_type=jnp.float32))
            e1 = jnp.exp(jnp.einsum("gid,gdj->gij", q1, k,
                                    preferred_element_type=jnp.float32))
            d0 = jnp.sum(e0, axis=-1, keepdims=True)
            d1 = jnp.sum(e1, axis=-1, keepdims=True)
            c0 = jnp.einsum("gij,gjd->gid", e0.astype(jnp.bfloat16), v0,
                            preferred_element_type=jnp.float32)
            c1 = jnp.einsum("gij,gjd->gid", e1.astype(jnp.bfloat16), v,
                            preferred_element_type=jnp.float32)
            ctx = jnp.concatenate([c0 / d0, c1 / d1], axis=1)
            ctx = ctx.astype(jnp.bfloat16).reshape(B_LOC, HQ_LOC, SQ, DH)
            ctx = ctx.transpose(0, 2, 1, 3).reshape(B_LOC * SQ, HD_LOC)
            return lax.dot_general(
                ctx, wot, (((1,), (1,)), ((), ())),
                preferred_element_type=jnp.float32,
            )

        def fwd(src, dst_buf, slot, send_sems, recv_sems, tgt):
            return pltpu.make_async_remote_copy(
                src_ref=src,
                dst_ref=dst_buf.at[slot],
                send_sem=send_sems.at[slot],
                recv_sem=recv_sems.at[slot],
                device_id=(tgt,),
                device_id_type=pl.DeviceIdType.MESH,
            )

        fwd(c0_ref, cw_buf, 0, cw_send, cw_recv, right).start()
        fwd(c0_ref, ccw_buf, 0, ccw_send, ccw_recv, left).start()
        acc_ref[...] = contribution(c0_ref[...], my_pos)

        def step(h, _):
            fwd(c0_ref, cw_buf, h, cw_send, cw_recv, right).wait_recv()

            @pl.when(h < N_CW - 1)
            def _():
                fwd(cw_buf.at[h], cw_buf, h + 1, cw_send, cw_recv,
                    right).start()

            fwd(c0_ref, ccw_buf, h, ccw_send, ccw_recv, left).wait_recv()

            @pl.when(h < N_CCW - 1)
            def _():
                fwd(ccw_buf.at[h], ccw_buf, h + 1, ccw_send, ccw_recv,
                    left).start()

            cw_origin = lax.rem(my_pos - 1 - h + 2 * N_DEV, N_DEV)
            ccw_origin = lax.rem(my_pos + 1 + h, N_DEV)
            acc_ref[...] += contribution(cw_buf[h], cw_origin)
            acc_ref[...] += contribution(ccw_buf[h], ccw_origin)

            fwd(cw_buf.at[h], cw_buf, h, cw_send, cw_recv, right).wait_send()
            fwd(ccw_buf.at[h], ccw_buf, h, ccw_send, ccw_recv,
                left).wait_send()
            return 0

        lax.fori_loop(0, N_CCW, step, 0)

        fwd(c0_ref, cw_buf, N_CW - 1, cw_send, cw_recv, right).wait_recv()
        fwd(cw_buf.at[N_CW - 1], cw_buf, N_CW - 1, cw_send, cw_recv,
            right).wait_send()
        far = lax.rem(my_pos + N_CW, N_DEV)
        acc_ref[...] += contribution(cw_buf[N_CW - 1], far)

        out_ref[...] = acc_ref[...].reshape(B_LOC, SQ, DM)

    return pl.pallas_call(
        body,
        out_shape=jax.ShapeDtypeStruct((B_LOC, SQ, DM), jnp.float32),
        in_specs=[
            pl.BlockSpec(memory_space=pltpu.VMEM),
            pl.BlockSpec(memory_space=pltpu.VMEM),
            pl.BlockSpec(memory_space=pltpu.VMEM),
            pl.BlockSpec(memory_space=pltpu.VMEM),
        ],
        out_specs=pl.BlockSpec(memory_space=pltpu.VMEM),
        scratch_shapes=[
            pltpu.VMEM((N_CW, DM, 2 * HD_LOC), jnp.bfloat16),
            pltpu.VMEM((N_CCW, DM, 2 * HD_LOC), jnp.bfloat16),
            pltpu.SemaphoreType.DMA((N_CW,)),
            pltpu.SemaphoreType.DMA((N_CW,)),
            pltpu.SemaphoreType.DMA((N_CCW,)),
            pltpu.SemaphoreType.DMA((N_CCW,)),
            pltpu.VMEM((B_LOC * SQ, DM), jnp.float32),
        ],
        compiler_params=pltpu.CompilerParams(collective_id=0),
    )(x_flat, c0, k_t, v_t)


# device time: 153970 ns/iter; 1.9702x vs baseline; 1.3998x over previous
import jax
import jax.numpy as jnp
from jax import lax
from jax.experimental import pallas as pl
from jax.experimental.pallas import tpu as pltpu

N_DEV = 32
N_CW = 16
N_CCW = 15
B_LOC = 2
HQ_LOC = 4
DH = 64
SQ = 128
SKV = 128
DM = 512
HD_LOC = HQ_LOC * DH
BLK = 64

PERM = (0, 3, 4, 7, 15, 12, 11, 8, 16, 19, 20, 23, 31, 28, 27, 24,
        25, 26, 29, 30, 22, 21, 18, 17, 9, 10, 13, 14, 6, 5, 2, 1)
INV = (0, 31, 30, 1, 2, 29, 28, 3, 7, 24, 25, 6, 5, 26, 27, 4,
       8, 23, 22, 9, 10, 21, 20, 11, 15, 16, 17, 14, 13, 18, 19, 12)


def kernel(x, Wq, K_ext, V_ext, Wo):
    my = lax.axis_index("i")

    kb = lax.dynamic_slice_in_dim(K_ext, my * B_LOC, B_LOC, axis=0)
    vb = lax.dynamic_slice_in_dim(V_ext, my * B_LOC, B_LOC, axis=0)
    k_t = jnp.transpose(kb.astype(jnp.bfloat16), (0, 2, 3, 1))
    v_t = jnp.transpose(vb.astype(jnp.bfloat16), (0, 2, 1, 3))
    x_flat = x.reshape(B_LOC * SQ, DM).astype(jnp.bfloat16)
    c0 = jnp.concatenate(
        [Wq.astype(jnp.bfloat16), Wo.astype(jnp.bfloat16).T], axis=1
    )

    perm = jnp.asarray(PERM, jnp.int32)
    r = jnp.take(jnp.asarray(INV, jnp.int32), my)
    nbrs = jnp.stack([
        jnp.take(perm, lax.rem(r + 1, N_DEV)),
        jnp.take(perm, lax.rem(r - 1 + N_DEV, N_DEV)),
    ]).astype(jnp.int32)
    org_cw = jnp.take(
        perm, lax.rem(r - 1 - jnp.arange(N_CW) + 2 * N_DEV, N_DEV)
    ).astype(jnp.int32)
    org_ccw = jnp.take(
        perm, lax.rem(r + 1 + jnp.arange(N_CCW), N_DEV)
    ).astype(jnp.int32)

    def body(nbrs_ref, org_cw_ref, org_ccw_ref, x_ref, c0_ref, k_ref, v_ref,
             out_ref, cw_buf, ccw_buf, cw_send, cw_recv, ccw_send, ccw_recv,
             acc_ref):
        my_pos = lax.axis_index("i")
        left = nbrs_ref[1]
        right = nbrs_ref[0]

        barrier_sem = pltpu.get_barrier_semaphore()
        for nbr in (left, right):
            pl.semaphore_signal(
                barrier_sem, inc=1,
                device_id=(nbr,), device_id_type=pl.DeviceIdType.MESH,
            )
        pl.semaphore_wait(barrier_sem, 2)

        def contribution(chunk, origin):
            wq = chunk[:, :HD_LOC]
            wot = chunk[:, HD_LOC:]
            q = jnp.dot(x_ref[...], wq, preferred_element_type=jnp.float32)
            q = (q * 0.125).astype(jnp.bfloat16)
            q = q.reshape(B_LOC, SQ, HQ_LOC, DH).transpose(0, 2, 1, 3)
            q = q.reshape(B_LOC * HQ_LOC, SQ, DH)
            h0 = origin * HQ_LOC
            k = k_ref[:, pl.ds(h0, HQ_LOC), :, :].reshape(
                B_LOC * HQ_LOC, DH, SKV)
            v = v_ref[:, pl.ds(h0, HQ_LOC), :, :].reshape(
                B_LOC * HQ_LOC, SKV, DH)
            q0 = q[:, :BLK, :]
            q1 = q[:, BLK:, :]
            k0 = k[:, :, :BLK]
            v0 = v[:, :BLK, :]
            e0 = jnp.exp(jnp.einsum("gid,gdj->gij", q0, k0,
                                    preferred_element_type=jnp.float32))
            e1 = jnp.exp(jnp.einsum("gid,gdj->gij", q1, k,
                                    preferred_element_type=jnp.float32))
            d0 = jnp.sum(e0, axis=-1, keepdims=True)
            d1 = jnp.sum(e1, axis=-1, keepdims=True)
            c0 = jnp.einsum("gij,gjd->gid", e0.astype(jnp.bfloat16), v0,
                            preferred_element_type=jnp.float32)
            c1 = jnp.einsum("gij,gjd->gid", e1.astype(jnp.bfloat16), v,
                            preferred_element_type=jnp.float32)
            ctx = jnp.concatenate([c0 / d0, c1 / d1], axis=1)
            ctx = ctx.astype(jnp.bfloat16).reshape(B_LOC, HQ_LOC, SQ, DH)
            ctx = ctx.transpose(0, 2, 1, 3).reshape(B_LOC * SQ, HD_LOC)
            return lax.dot_general(
                ctx, wot, (((1,), (1,)), ((), ())),
                preferred_element_type=jnp.float32,
            )

        def fwd(src, dst_buf, slot, send_sems, recv_sems, tgt):
            return pltpu.make_async_remote_copy(
                src_ref=src,
                dst_ref=dst_buf.at[slot],
                send_sem=send_sems.at[slot],
                recv_sem=recv_sems.at[slot],
                device_id=(tgt,),
                device_id_type=pl.DeviceIdType.MESH,
            )

        fwd(c0_ref, cw_buf, 0, cw_send, cw_recv, right).start()
        fwd(c0_ref, ccw_buf, 0, ccw_send, ccw_recv, left).start()
        acc_ref[...] = contribution(c0_ref[...], my_pos)

        def step(h, _):
            fwd(c0_ref, cw_buf, h, cw_send, cw_recv, right).wait_recv()

            @pl.when(h < N_CW - 1)
            def _():
                fwd(cw_buf.at[h], cw_buf, h + 1, cw_send, cw_recv,
                    right).start()

            fwd(c0_ref, ccw_buf, h, ccw_send, ccw_recv, left).wait_recv()

            @pl.when(h < N_CCW - 1)
            def _():
                fwd(ccw_buf.at[h], ccw_buf, h + 1, ccw_send, ccw_recv,
                    left).start()

            cw_origin = org_cw_ref[h]
            ccw_origin = org_ccw_ref[h]
            acc_ref[...] += contribution(cw_buf[h], cw_origin)
            acc_ref[...] += contribution(ccw_buf[h], ccw_origin)

            fwd(cw_buf.at[h], cw_buf, h, cw_send, cw_recv, right).wait_send()
            fwd(ccw_buf.at[h], ccw_buf, h, ccw_send, ccw_recv,
                left).wait_send()
            return 0

        lax.fori_loop(0, N_CCW, step, 0)

        fwd(c0_ref, cw_buf, N_CW - 1, cw_send, cw_recv, right).wait_recv()
        fwd(cw_buf.at[N_CW - 1], cw_buf, N_CW - 1, cw_send, cw_recv,
            right).wait_send()
        acc_ref[...] += contribution(cw_buf[N_CW - 1], org_cw_ref[N_CW - 1])

        out_ref[...] = acc_ref[...].reshape(B_LOC, SQ, DM)

    return pl.pallas_call(
        body,
        out_shape=jax.ShapeDtypeStruct((B_LOC, SQ, DM), jnp.float32),
        in_specs=[
            pl.BlockSpec(memory_space=pltpu.SMEM),
            pl.BlockSpec(memory_space=pltpu.SMEM),
            pl.BlockSpec(memory_space=pltpu.SMEM),
            pl.BlockSpec(memory_space=pltpu.VMEM),
            pl.BlockSpec(memory_space=pltpu.VMEM),
            pl.BlockSpec(memory_space=pltpu.VMEM),
            pl.BlockSpec(memory_space=pltpu.VMEM),
        ],
        out_specs=pl.BlockSpec(memory_space=pltpu.VMEM),
        scratch_shapes=[
            pltpu.VMEM((N_CW, DM, 2 * HD_LOC), jnp.bfloat16),
            pltpu.VMEM((N_CCW, DM, 2 * HD_LOC), jnp.bfloat16),
            pltpu.SemaphoreType.DMA((N_CW,)),
            pltpu.SemaphoreType.DMA((N_CW,)),
            pltpu.SemaphoreType.DMA((N_CCW,)),
            pltpu.SemaphoreType.DMA((N_CCW,)),
            pltpu.VMEM((B_LOC * SQ, DM), jnp.float32),
        ],
        compiler_params=pltpu.CompilerParams(collective_id=0),
    )(nbrs, org_cw, org_ccw, x_flat, c0, k_t, v_t)


# device time: 127765 ns/iter; 2.3743x vs baseline; 1.2051x over previous
import jax
import jax.numpy as jnp
from jax import lax
from jax.experimental import pallas as pl
from jax.experimental.pallas import tpu as pltpu

N_DEV = 32
N_CW = 16
N_CCW = 15
B_LOC = 2
HQ_LOC = 4
DH = 64
SQ = 128
SKV = 128
DM = 512
HD_LOC = HQ_LOC * DH
BLK = 64
HROWS = DM // 2

PERM = (0, 3, 4, 7, 15, 12, 11, 8, 16, 19, 20, 23, 31, 28, 27, 24,
        25, 26, 29, 30, 22, 21, 18, 17, 9, 10, 13, 14, 6, 5, 2, 1)
INV = (0, 31, 30, 1, 2, 29, 28, 3, 7, 24, 25, 6, 5, 26, 27, 4,
       8, 23, 22, 9, 10, 21, 20, 11, 15, 16, 17, 14, 13, 18, 19, 12)


def kernel(x, Wq, K_ext, V_ext, Wo):
    my = lax.axis_index("i")

    kb = lax.dynamic_slice_in_dim(K_ext, my * B_LOC, B_LOC, axis=0)
    vb = lax.dynamic_slice_in_dim(V_ext, my * B_LOC, B_LOC, axis=0)
    k_t = jnp.transpose(kb.astype(jnp.bfloat16), (0, 2, 3, 1))
    v_t = jnp.transpose(vb.astype(jnp.bfloat16), (0, 2, 1, 3))
    x_flat = x.reshape(B_LOC * SQ, DM).astype(jnp.bfloat16)
    c0 = jnp.concatenate(
        [(Wq * 0.125).astype(jnp.bfloat16), Wo.astype(jnp.bfloat16).T],
        axis=1,
    ).reshape(2, HROWS, 2 * HD_LOC)

    perm = jnp.asarray(PERM, jnp.int32)
    r = jnp.take(jnp.asarray(INV, jnp.int32), my)
    nbrs = jnp.stack([
        jnp.take(perm, lax.rem(r + 1, N_DEV)),
        jnp.take(perm, lax.rem(r - 1 + N_DEV, N_DEV)),
    ]).astype(jnp.int32)
    org_cw = jnp.take(
        perm, lax.rem(r - 1 - jnp.arange(N_CW) + 2 * N_DEV, N_DEV)
    ).astype(jnp.int32)
    org_ccw = jnp.take(
        perm, lax.rem(r + 1 + jnp.arange(N_CCW), N_DEV)
    ).astype(jnp.int32)

    def body(nbrs_ref, org_cw_ref, org_ccw_ref, x_ref, c0_ref, k_ref, v_ref,
             out_ref, cw_buf, ccw_buf, cw_send, cw_recv, ccw_send, ccw_recv,
             acc_ref):
        my_pos = lax.axis_index("i")
        left = nbrs_ref[1]
        right = nbrs_ref[0]

        barrier_sem = pltpu.get_barrier_semaphore()
        for nbr in (left, right):
            pl.semaphore_signal(
                barrier_sem, inc=1,
                device_id=(nbr,), device_id_type=pl.DeviceIdType.MESH,
            )
        pl.semaphore_wait(barrier_sem, 2)

        def contribution(chunk, origin):
            wq = chunk[:, :HD_LOC]
            wot = chunk[:, HD_LOC:]
            q = jnp.dot(x_ref[...], wq,
                        preferred_element_type=jnp.float32)
            q = q.astype(jnp.bfloat16)
            q = q.reshape(B_LOC, SQ, HQ_LOC, DH).transpose(0, 2, 1, 3)
            q = q.reshape(B_LOC * HQ_LOC, SQ, DH)
            h0 = origin * HQ_LOC
            k = k_ref[:, pl.ds(h0, HQ_LOC), :, :].reshape(
                B_LOC * HQ_LOC, DH, SKV)
            v = v_ref[:, pl.ds(h0, HQ_LOC), :, :].reshape(
                B_LOC * HQ_LOC, SKV, DH)
            q0 = q[:, :BLK, :]
            q1 = q[:, BLK:, :]
            k0 = k[:, :, :BLK]
            v0 = v[:, :BLK, :]
            e0 = jnp.exp(jnp.einsum("gid,gdj->gij", q0, k0,
                                    preferred_element_type=jnp.float32))
            e1 = jnp.exp(jnp.einsum("gid,gdj->gij", q1, k,
                                    preferred_element_type=jnp.float32))
            d0 = jnp.sum(e0, axis=-1, keepdims=True)
            d1 = jnp.sum(e1, axis=-1, keepdims=True)
            c0b = jnp.einsum("gij,gjd->gid", e0.astype(jnp.bfloat16), v0,
                             preferred_element_type=jnp.float32)
            c1b = jnp.einsum("gij,gjd->gid", e1.astype(jnp.bfloat16), v,
                             preferred_element_type=jnp.float32)
            ctx = jnp.concatenate([c0b / d0, c1b / d1], axis=1)
            ctx = ctx.astype(jnp.bfloat16).reshape(B_LOC, HQ_LOC, SQ, DH)
            ctx = ctx.transpose(0, 2, 1, 3).reshape(B_LOC * SQ, HD_LOC)
            return lax.dot_general(
                ctx, wot, (((1,), (1,)), ((), ())),
                preferred_element_type=jnp.float32,
            )

        def fwd(src, dst_buf, slot, hf, send_sems, recv_sems, tgt):
            return pltpu.make_async_remote_copy(
                src_ref=src,
                dst_ref=dst_buf.at[slot, hf],
                send_sem=send_sems.at[slot, hf],
                recv_sem=recv_sems.at[slot, hf],
                device_id=(tgt,),
                device_id_type=pl.DeviceIdType.MESH,
            )

        fwd(c0_ref.at[0], cw_buf, 0, 0, cw_send, cw_recv, right).start()
        fwd(c0_ref.at[0], ccw_buf, 0, 0, ccw_send, ccw_recv, left).start()
        fwd(c0_ref.at[1], cw_buf, 0, 1, cw_send, cw_recv, right).start()
        fwd(c0_ref.at[1], ccw_buf, 0, 1, ccw_send, ccw_recv, left).start()
        acc_ref[...] = contribution(
            c0_ref[...].reshape(DM, 2 * HD_LOC), my_pos)

        def step(h, _):
            fwd(c0_ref.at[0], cw_buf, h, 0, cw_send, cw_recv,
                right).wait_recv()

            @pl.when(h < N_CW - 1)
            def _():
                fwd(cw_buf.at[h, 0], cw_buf, h + 1, 0, cw_send, cw_recv,
                    right).start()

            fwd(c0_ref.at[0], ccw_buf, h, 0, ccw_send, ccw_recv,
                left).wait_recv()

            @pl.when(h < N_CCW - 1)
            def _():
                fwd(ccw_buf.at[h, 0], ccw_buf, h + 1, 0, ccw_send, ccw_recv,
                    left).start()

            fwd(c0_ref.at[1], cw_buf, h, 1, cw_send, cw_recv,
                right).wait_recv()

            @pl.when(h < N_CW - 1)
            def _():
                fwd(cw_buf.at[h, 1], cw_buf, h + 1, 1, cw_send, cw_recv,
                    right).start()

            fwd(c0_ref.at[1], ccw_buf, h, 1, ccw_send, ccw_recv,
                left).wait_recv()

            @pl.when(h < N_CCW - 1)
            def _():
                fwd(ccw_buf.at[h, 1], ccw_buf, h + 1, 1, ccw_send, ccw_recv,
                    left).start()

            acc_ref[...] += contribution(
                cw_buf[h].reshape(DM, 2 * HD_LOC), org_cw_ref[h])
            acc_ref[...] += contribution(
                ccw_buf[h].reshape(DM, 2 * HD_LOC), org_ccw_ref[h])

            for hf in (0, 1):
                fwd(cw_buf.at[h, hf], cw_buf, h, hf, cw_send, cw_recv,
                    right).wait_send()
                fwd(ccw_buf.at[h, hf], ccw_buf, h, hf, ccw_send, ccw_recv,
                    left).wait_send()
            return 0

        lax.fori_loop(0, N_CCW, step, 0)

        for hf in (0, 1):
            fwd(c0_ref.at[hf], cw_buf, N_CW - 1, hf, cw_send, cw_recv,
                right).wait_recv()
            fwd(cw_buf.at[N_CW - 1, hf], cw_buf, N_CW - 1, hf, cw_send,
                cw_recv, right).wait_send()
        acc_ref[...] += contribution(
            cw_buf[N_CW - 1].reshape(DM, 2 * HD_LOC), org_cw_ref[N_CW - 1])

        out_ref[...] = acc_ref[...].reshape(B_LOC, SQ, DM)

    return pl.pallas_call(
        body,
        out_shape=jax.ShapeDtypeStruct((B_LOC, SQ, DM), jnp.float32),
        in_specs=[
            pl.BlockSpec(memory_space=pltpu.SMEM),
            pl.BlockSpec(memory_space=pltpu.SMEM),
            pl.BlockSpec(memory_space=pltpu.SMEM),
            pl.BlockSpec(memory_space=pltpu.VMEM),
            pl.BlockSpec(memory_space=pltpu.VMEM),
            pl.BlockSpec(memory_space=pltpu.VMEM),
            pl.BlockSpec(memory_space=pltpu.VMEM),
        ],
        out_specs=pl.BlockSpec(memory_space=pltpu.VMEM),
        scratch_shapes=[
            pltpu.VMEM((N_CW, 2, HROWS, 2 * HD_LOC), jnp.bfloat16),
            pltpu.VMEM((N_CCW, 2, HROWS, 2 * HD_LOC), jnp.bfloat16),
            pltpu.SemaphoreType.DMA((N_CW, 2)),
            pltpu.SemaphoreType.DMA((N_CW, 2)),
            pltpu.SemaphoreType.DMA((N_CCW, 2)),
            pltpu.SemaphoreType.DMA((N_CCW, 2)),
            pltpu.VMEM((B_LOC * SQ, DM), jnp.float32),
        ],
        compiler_params=pltpu.CompilerParams(collective_id=0),
    )(nbrs, org_cw, org_ccw, x_flat, c0, k_t, v_t)
